# Initial kernel scaffold; baseline (speedup 1.0000x reference)
import jax
import jax.numpy as jnp
from jax.experimental import pallas as pl


def kernel(batch_token, edge_p_node, edge_c_node, edge_p_indicate, edge_c_indicate, p_mask, c_mask, start_token, end_token, params):
    raise NotImplementedError("write your pallas kernel here")


# speedup vs baseline: 2.0244x; 2.0244x over previous
import functools

import jax
import jax.numpy as jnp
from jax import lax
from jax.experimental import pallas as pl
from jax.experimental.pallas import tpu as pltpu
from jax.experimental.pallas import tpu_sc as plsc

F32 = jnp.float32

N_NODES = 10000
N_EDGES = 320000
EMBED = 128
H1 = 256
H2 = 128

N_PAD = 10240
E_PAD = 327680
E2 = 2 * E_PAD

BN = 1024
BE = 2560

NC, NS = 2, 16
NW = NC * NS
CH = 512
CHI = CH // 128
SUP = 1024
SUPI = SUP // 128
SCH = 256
SCHI = SCH // 128


def _ln_relu(h, g, b):
    m = jnp.mean(h, axis=-1, keepdims=True)
    v = jnp.mean((h - m) ** 2, axis=-1, keepdims=True)
    return jnp.maximum((h - m) * lax.rsqrt(v + 1e-5) * g + b, 0.0)


def _dot(a, b):
    return lax.dot_general(a, b, (((1,), (0,)), ((), ())),
                           preferred_element_type=F32)


def _mlp2_body(x, w1t, b1, g1, be1, w2t, b2, g2, be2, o):
    h = _dot(x[...], w1t[...]) + b1[...]
    h = _ln_relu(h, g1[...], be1[...])
    h = _dot(h, w2t[...]) + b2[...]
    o[...] = _ln_relu(h, g2[...], be2[...])


def _emlp_body(x, w1r, b1, g1, be1, w2t, b2, g2, be2, o):
    h = x[...] * w1r[...] + b1[...]
    h = _ln_relu(h, g1[...], be1[...])
    h = _dot(h, w2t[...]) + b2[...]
    o[...] = _ln_relu(h, g2[...], be2[...])


def _edge_body(pb, cb, ep, ec,
               pat, pbt, pct, pb1, pg1, pbe1, pw2, pb2, pg2, pbe2,
               cat, cbt, cct, cb1, cg1, cbe1, cw2, cb2, cg2, cbe2,
               sp, sc):
    pbv, cbv = pb[...], cb[...]
    h = _dot(cbv, pat[...]) + _dot(pbv, pbt[...]) + _dot(ep[...], pct[...]) + pb1[...]
    h = _ln_relu(h, pg1[...], pbe1[...])
    h = _dot(h, pw2[...]) + pb2[...]
    sp[...] = _ln_relu(h, pg2[...], pbe2[...])

    h = _dot(pbv, cat[...]) + _dot(cbv, cbt[...]) + _dot(ec[...], cct[...]) + cb1[...]
    h = _ln_relu(h, cg1[...], cbe1[...])
    h = _dot(h, cw2[...]) + cb2[...]
    sc[...] = _ln_relu(h, cg2[...], cbe2[...])


def _update_body(h, sps, scs, cp, cc, pm, cm, st, en,
                 at_, bt_, ct_, b1, g1, be1, w2t, b2, g2, be2, o):
    hv = h[...]
    sp = sps[...] / jnp.maximum(cp[...][:, 0:1], 1.0) + pm[...] * st[...]
    sc = scs[...] / jnp.maximum(cc[...][:, 0:1], 1.0) + cm[...] * en[...]
    x = _dot(hv, at_[...]) + _dot(sp, bt_[...]) + _dot(sc, ct_[...]) + b1[...]
    x = _ln_relu(x, g1[...], be1[...])
    x = _dot(x, w2t[...]) + b2[...]
    x = _ln_relu(x, g2[...], be2[...])
    o[...] = jnp.maximum(hv + x, 0.0)


def _full(shape):
    return pl.BlockSpec(shape, lambda i: (0, 0))


_CPARAMS = pltpu.CompilerParams(dimension_semantics=("arbitrary",))


def _node_mlp(x, w):
    specs = [pl.BlockSpec((BN, EMBED), lambda i: (i, 0))]
    specs += [_full(t.shape) for t in w]
    return pl.pallas_call(
        _mlp2_body,
        grid=(N_PAD // BN,),
        in_specs=specs,
        out_specs=pl.BlockSpec((BN, H2), lambda i: (i, 0)),
        out_shape=jax.ShapeDtypeStruct((N_PAD, H2), F32),
        compiler_params=_CPARAMS,
    )(x, *w)


def _edge_ind_mlp(x, w):
    specs = [pl.BlockSpec((BE, 1), lambda i: (i, 0))]
    specs += [_full(t.shape) for t in w]
    return pl.pallas_call(
        _emlp_body,
        grid=(E2 // BE,),
        in_specs=specs,
        out_specs=pl.BlockSpec((BE, H2), lambda i: (i, 0)),
        out_shape=jax.ShapeDtypeStruct((E2, H2), F32),
        compiler_params=_CPARAMS,
    )(x, *w)


def _edge_mlp(pb, cb, ep, ec, wp, wc):
    dspec = pl.BlockSpec((BE, H2), lambda i: (i, 0))
    specs = [dspec] * 4 + [_full(t.shape) for t in (*wp, *wc)]
    return pl.pallas_call(
        _edge_body,
        grid=(E_PAD // BE,),
        in_specs=specs,
        out_specs=[dspec, dspec],
        out_shape=[jax.ShapeDtypeStruct((E_PAD, H2), F32)] * 2,
        compiler_params=_CPARAMS,
    )(pb, cb, ep, ec, *wp, *wc)


def _node_update(h, sps, scs, cp, cc, pm, cm, st, en, wa):
    nspec = pl.BlockSpec((BN, H2), lambda i: (i, 0))
    cspec = pl.BlockSpec((BN, H2), lambda i: (i, 0))
    mspec = pl.BlockSpec((BN, 1), lambda i: (i, 0))
    tspec = _full((1, H2))
    specs = [nspec, nspec, nspec, cspec, cspec, mspec, mspec, tspec, tspec]
    specs += [_full(t.shape) for t in wa]
    return pl.pallas_call(
        _update_body,
        grid=(N_PAD // BN,),
        in_specs=specs,
        out_specs=nspec,
        out_shape=jax.ShapeDtypeStruct((N_PAD, H2), F32),
        compiler_params=_CPARAMS,
    )(h, sps, scs, cp, cc, pm, cm, st, en, *wa)


_G_PER_W = E2 // NW
_S_PER_S = E_PAD // NS


@functools.cache
def _mesh():
    return plsc.VectorSubcoreMesh(core_axis_name="c", subcore_axis_name="s")


def _sc_gather(h, idx2d):
    @functools.partial(
        pl.kernel, mesh=_mesh(),
        out_type=jax.ShapeDtypeStruct((E2, H2), F32),
        scratch_types=[pltpu.VMEM((SUPI, 128), jnp.int32),
                       pltpu.VMEM((CH, H2), F32),
                       pltpu.SemaphoreType.DMA],
    )
    def _gather_k(h_hbm, idx_hbm, out_hbm, idx_v, rows_v, sem):
        wid = lax.axis_index("s") * NC + lax.axis_index("c")
        base = wid * _G_PER_W

        @pl.loop(0, _G_PER_W // SUP)
        def _(i):
            b = base + i * SUP
            pltpu.sync_copy(
                idx_hbm.at[pl.ds(pl.multiple_of(b // 128, 8), SUPI), :], idx_v)
            for half in range(SUP // CH):
                cps = [pltpu.async_copy(h_hbm.at[idx_v.at[half * CHI + j]],
                                        rows_v.at[pl.ds(j * 128, 128)], sem)
                       for j in range(CHI)]
                for c in cps:
                    c.wait()
                pltpu.sync_copy(
                    rows_v,
                    out_hbm.at[pl.ds(pl.multiple_of(b + half * CH, 8), CH)])

    return _gather_k(h, idx2d)


def _sc_scatter(sp, sc, idxp2d, idxc2d, zeros):
    @functools.partial(
        pl.kernel, mesh=_mesh(),
        out_type=jax.ShapeDtypeStruct((NC, N_PAD, H2), F32),
        scratch_types=[pltpu.VMEM((SUPI, 128), jnp.int32),
                       pltpu.VMEM((SCH, H2), F32),
                       pltpu.VMEM_SHARED((N_PAD, H2), F32),
                       pltpu.SemaphoreType.DMA],
    )
    def _scatter_k(sp_hbm, sc_hbm, idxp_hbm, idxc_hbm, zeros_hbm, out_hbm,
                   idx_v, val_v, acc_sh, sem):
        c = lax.axis_index("c")
        s = lax.axis_index("s")

        @pl.when(s == 0)
        def _():
            pltpu.sync_copy(zeros_hbm, acc_sh)

        plsc.subcore_barrier()
        base = s * _S_PER_S

        def _accum(val_hbm, i_hbm):
            @pl.loop(0, _S_PER_S // SUP)
            def _(i):
                b = base + i * SUP
                pltpu.sync_copy(
                    i_hbm.at[pl.ds(pl.multiple_of(b // 128, 8), SUPI), :],
                    idx_v)
                for half in range(SUP // SCH):
                    pltpu.sync_copy(
                        val_hbm.at[pl.ds(pl.multiple_of(b + half * SCH, 8), SCH)],
                        val_v)
                    for j in range(SCHI):
                        pltpu.sync_copy(val_v.at[pl.ds(j * 128, 128)],
                                        acc_sh.at[idx_v.at[half * SCHI + j]],
                                        add=True)

        @pl.when(c == 0)
        def _():
            _accum(sp_hbm, idxp_hbm)

        @pl.when(c == 1)
        def _():
            _accum(sc_hbm, idxc_hbm)

        plsc.subcore_barrier()

        @pl.when(s == 0)
        def _():
            pltpu.sync_copy(acc_sh, out_hbm.at[c])

    return _scatter_k(sp, sc, idxp2d, idxc2d, zeros)


def _sc_counts(idxp2d, idxc2d, ones, zeros):
    @functools.partial(
        pl.kernel, mesh=_mesh(),
        out_type=jax.ShapeDtypeStruct((NC, N_PAD, H2), F32),
        scratch_types=[pltpu.VMEM((SUPI, 128), jnp.int32),
                       pltpu.VMEM((128, H2), F32),
                       pltpu.VMEM_SHARED((N_PAD, H2), F32),
                       pltpu.SemaphoreType.DMA],
    )
    def _counts_k(idxp_hbm, idxc_hbm, ones_hbm, zeros_hbm, out_hbm,
                  idx_v, ones_v, acc_sh, sem):
        c = lax.axis_index("c")
        s = lax.axis_index("s")

        pltpu.sync_copy(ones_hbm, ones_v)

        @pl.when(s == 0)
        def _():
            pltpu.sync_copy(zeros_hbm, acc_sh)

        plsc.subcore_barrier()
        base = s * _S_PER_S

        def _accum(i_hbm):
            @pl.loop(0, _S_PER_S // SUP)
            def _(i):
                b = base + i * SUP
                pltpu.sync_copy(
                    i_hbm.at[pl.ds(pl.multiple_of(b // 128, 8), SUPI), :],
                    idx_v)
                for j in range(SUPI):
                    pltpu.sync_copy(ones_v, acc_sh.at[idx_v.at[j]], add=True)

        @pl.when(c == 0)
        def _():
            _accum(idxp_hbm)

        @pl.when(c == 1)
        def _():
            _accum(idxc_hbm)

        plsc.subcore_barrier()

        @pl.when(s == 0)
        def _():
            pltpu.sync_copy(acc_sh, out_hbm.at[c])

    return _counts_k(idxp2d, idxc2d, ones, zeros)


def _row(v):
    return v.reshape(1, -1).astype(F32)


def _prep2(p):
    return (p['W1'].T.astype(F32), _row(p['b1']), _row(p['g1']), _row(p['be1']),
            p['W2'].T.astype(F32), _row(p['b2']), _row(p['g2']), _row(p['be2']))


def _prep_e(p):
    return (p['W1'][:, 0].reshape(1, -1).astype(F32), _row(p['b1']),
            _row(p['g1']), _row(p['be1']),
            p['W2'].T.astype(F32), _row(p['b2']), _row(p['g2']), _row(p['be2']))


def _prep3(p):
    w1 = p['W1'].astype(F32)
    return (w1[:, :H2].T, w1[:, H2:2 * H2].T, w1[:, 2 * H2:].T,
            _row(p['b1']), _row(p['g1']), _row(p['be1']),
            p['W2'].T.astype(F32), _row(p['b2']), _row(p['g2']), _row(p['be2']))


def kernel(batch_token, edge_p_node, edge_c_node, edge_p_indicate,
           edge_c_indicate, p_mask, c_mask, start_token, end_token, params):
    bt = jnp.zeros((N_PAD, EMBED), F32).at[:N_NODES].set(batch_token)
    idx_p = jnp.full((E_PAD,), N_NODES, jnp.int32).at[:N_EDGES].set(
        edge_p_node.astype(jnp.int32))
    idx_c = jnp.full((E_PAD,), N_NODES, jnp.int32).at[:N_EDGES].set(
        edge_c_node.astype(jnp.int32))
    idx_all2d = jnp.concatenate([idx_p, idx_c]).reshape(E2 // 128, 128)
    idx_p2d = idx_p.reshape(E_PAD // 128, 128)
    idx_c2d = idx_c.reshape(E_PAD // 128, 128)

    ind = jnp.zeros((E2, 1), F32)
    ind = ind.at[:N_EDGES, 0].set(edge_p_indicate)
    ind = ind.at[E_PAD:E_PAD + N_EDGES, 0].set(edge_c_indicate)

    pm = jnp.zeros((N_PAD, 1), F32).at[:N_NODES, 0].set(p_mask)
    cm = jnp.zeros((N_PAD, 1), F32).at[:N_NODES, 0].set(c_mask)
    st = start_token.reshape(1, H2).astype(F32)
    en = end_token.reshape(1, H2).astype(F32)

    zN = jnp.zeros((N_PAD, H2), F32)
    ones128 = jnp.ones((128, H2), F32)

    wV = _prep2(params['V'])
    wE = _prep_e(params['E'])

    h = _node_mlp(bt, wV)
    ef = _edge_ind_mlp(ind, wE)
    ep, ec = ef[:E_PAD], ef[E_PAD:]

    counts = _sc_counts(idx_p2d, idx_c2d, ones128, zN)
    cp, cc = counts[0], counts[1]

    for hop in range(2):
        wp = _prep3(params['P'][hop])
        wc = _prep3(params['C'][hop])
        wa = _prep3(params['A'][hop])

        gath = _sc_gather(h, idx_all2d)
        pb, cb = gath[:E_PAD], gath[E_PAD:]
        sp, sc = _edge_mlp(pb, cb, ep, ec, wp, wc)
        sums = _sc_scatter(sp, sc, idx_p2d, idx_c2d, zN)
        h = _node_update(h, sums[0], sums[1], cp, cc, pm, cm, st, en, wa)

    return h[:N_NODES]


# speedup vs baseline: 2.0425x; 1.0089x over previous
import functools

import jax
import jax.numpy as jnp
from jax import lax
from jax.experimental import pallas as pl
from jax.experimental.pallas import tpu as pltpu
from jax.experimental.pallas import tpu_sc as plsc

F32 = jnp.float32
BF16 = jnp.bfloat16

N_NODES = 10000
N_EDGES = 320000
EMBED = 128
H1 = 256
H2 = 128

N_PAD = 10240
E_PAD = 327680
E2 = 2 * E_PAD

BN = 1024
BE = 2560

NC, NS = 2, 16
NW = NC * NS
CH = 512
CHI = CH // 128
SUP = 1024
SUPI = SUP // 128
SCH = 256
SCHI = SCH // 128


def _ln_relu(h, g, b):
    m = jnp.mean(h, axis=-1, keepdims=True)
    v = jnp.mean((h - m) ** 2, axis=-1, keepdims=True)
    return jnp.maximum((h - m) * lax.rsqrt(v + 1e-5) * g + b, 0.0)


def _dot(a, b):
    return lax.dot_general(a, b, (((1,), (0,)), ((), ())),
                           preferred_element_type=F32)


def _mlp2_body(x, w1t, b1, g1, be1, w2t, b2, g2, be2, o):
    h = _dot(x[...], w1t[...]) + b1[...]
    h = _ln_relu(h, g1[...], be1[...])
    h = _dot(h, w2t[...]) + b2[...]
    o[...] = _ln_relu(h, g2[...], be2[...])


def _emlp_body(x, w1r, b1, g1, be1, w2t, b2, g2, be2, o):
    h = x[...] * w1r[...] + b1[...]
    h = _ln_relu(h, g1[...], be1[...])
    h = _dot(h, w2t[...]) + b2[...]
    o[...] = _ln_relu(h, g2[...], be2[...]).astype(BF16)


def _edge_body(pb, cb, ep, ec,
               pat, pbt, pct, pb1, pg1, pbe1, pw2, pb2, pg2, pbe2,
               cat, cbt, cct, cb1, cg1, cbe1, cw2, cb2, cg2, cbe2,
               sp, sc):
    pbv, cbv = pb[...].astype(BF16), cb[...].astype(BF16)
    h = _dot(cbv, pat[...]) + _dot(pbv, pbt[...]) + _dot(ep[...], pct[...]) + pb1[...]
    h = _ln_relu(h, pg1[...], pbe1[...]).astype(BF16)
    h = _dot(h, pw2[...]) + pb2[...]
    sp[...] = _ln_relu(h, pg2[...], pbe2[...])

    h = _dot(pbv, cat[...]) + _dot(cbv, cbt[...]) + _dot(ec[...], cct[...]) + cb1[...]
    h = _ln_relu(h, cg1[...], cbe1[...]).astype(BF16)
    h = _dot(h, cw2[...]) + cb2[...]
    sc[...] = _ln_relu(h, cg2[...], cbe2[...])


def _update_body(h, sps, scs, cp, cc, pm, cm, st, en,
                 at_, bt_, ct_, b1, g1, be1, w2t, b2, g2, be2, o):
    hv = h[...]
    sp = sps[...] / jnp.maximum(cp[...][:, 0:1], 1.0) + pm[...] * st[...]
    sc = scs[...] / jnp.maximum(cc[...][:, 0:1], 1.0) + cm[...] * en[...]
    x = _dot(hv, at_[...]) + _dot(sp, bt_[...]) + _dot(sc, ct_[...]) + b1[...]
    x = _ln_relu(x, g1[...], be1[...])
    x = _dot(x, w2t[...]) + b2[...]
    x = _ln_relu(x, g2[...], be2[...])
    o[...] = jnp.maximum(hv + x, 0.0)


def _full(shape):
    return pl.BlockSpec(shape, lambda i: (0, 0))


_CPARAMS = pltpu.CompilerParams(dimension_semantics=("arbitrary",))


def _node_mlp(x, w):
    specs = [pl.BlockSpec((BN, EMBED), lambda i: (i, 0))]
    specs += [_full(t.shape) for t in w]
    return pl.pallas_call(
        _mlp2_body,
        grid=(N_PAD // BN,),
        in_specs=specs,
        out_specs=pl.BlockSpec((BN, H2), lambda i: (i, 0)),
        out_shape=jax.ShapeDtypeStruct((N_PAD, H2), F32),
        compiler_params=_CPARAMS,
    )(x, *w)


def _edge_ind_mlp(x, w):
    specs = [pl.BlockSpec((BE, 1), lambda i: (i, 0))]
    specs += [_full(t.shape) for t in w]
    return pl.pallas_call(
        _emlp_body,
        grid=(E2 // BE,),
        in_specs=specs,
        out_specs=pl.BlockSpec((BE, H2), lambda i: (i, 0)),
        out_shape=jax.ShapeDtypeStruct((E2, H2), BF16),
        compiler_params=_CPARAMS,
    )(x, *w)


def _edge_mlp(pb, cb, ep, ec, wp, wc):
    dspec = pl.BlockSpec((BE, H2), lambda i: (i, 0))
    specs = [dspec] * 4 + [_full(t.shape) for t in (*wp, *wc)]
    return pl.pallas_call(
        _edge_body,
        grid=(E_PAD // BE,),
        in_specs=specs,
        out_specs=[dspec, dspec],
        out_shape=[jax.ShapeDtypeStruct((E_PAD, H2), F32)] * 2,
        compiler_params=_CPARAMS,
    )(pb, cb, ep, ec, *wp, *wc)


def _node_update(h, sps, scs, cp, cc, pm, cm, st, en, wa):
    nspec = pl.BlockSpec((BN, H2), lambda i: (i, 0))
    cspec = pl.BlockSpec((BN, H2), lambda i: (i, 0))
    mspec = pl.BlockSpec((BN, 1), lambda i: (i, 0))
    tspec = _full((1, H2))
    specs = [nspec, nspec, nspec, cspec, cspec, mspec, mspec, tspec, tspec]
    specs += [_full(t.shape) for t in wa]
    return pl.pallas_call(
        _update_body,
        grid=(N_PAD // BN,),
        in_specs=specs,
        out_specs=nspec,
        out_shape=jax.ShapeDtypeStruct((N_PAD, H2), F32),
        compiler_params=_CPARAMS,
    )(h, sps, scs, cp, cc, pm, cm, st, en, *wa)


_G_PER_W = E2 // NW
_S_PER_S = E_PAD // NS


@functools.cache
def _mesh():
    return plsc.VectorSubcoreMesh(core_axis_name="c", subcore_axis_name="s")


def _sc_gather(h, idx2d):
    @functools.partial(
        pl.kernel, mesh=_mesh(),
        out_type=jax.ShapeDtypeStruct((E2, H2), F32),
        scratch_types=[pltpu.VMEM((SUPI, 128), jnp.int32),
                       pltpu.VMEM((CH, H2), F32),
                       pltpu.SemaphoreType.DMA],
    )
    def _gather_k(h_hbm, idx_hbm, out_hbm, idx_v, rows_v, sem):
        wid = lax.axis_index("s") * NC + lax.axis_index("c")
        base = wid * _G_PER_W

        @pl.loop(0, _G_PER_W // SUP)
        def _(i):
            b = base + i * SUP
            pltpu.sync_copy(
                idx_hbm.at[pl.ds(pl.multiple_of(b // 128, 8), SUPI), :], idx_v)
            for half in range(SUP // CH):
                cps = [pltpu.async_copy(h_hbm.at[idx_v.at[half * CHI + j]],
                                        rows_v.at[pl.ds(j * 128, 128)], sem)
                       for j in range(CHI)]
                for c in cps:
                    c.wait()
                pltpu.sync_copy(
                    rows_v,
                    out_hbm.at[pl.ds(pl.multiple_of(b + half * CH, 8), CH)])

    return _gather_k(h, idx2d)


def _sc_scatter(sp, sc, idxp2d, idxc2d, zeros):
    @functools.partial(
        pl.kernel, mesh=_mesh(),
        out_type=jax.ShapeDtypeStruct((NC, N_PAD, H2), F32),
        scratch_types=[pltpu.VMEM((SUPI, 128), jnp.int32),
                       pltpu.VMEM((SCH, H2), F32),
                       pltpu.VMEM_SHARED((N_PAD, H2), F32),
                       pltpu.SemaphoreType.DMA],
    )
    def _scatter_k(sp_hbm, sc_hbm, idxp_hbm, idxc_hbm, zeros_hbm, out_hbm,
                   idx_v, val_v, acc_sh, sem):
        c = lax.axis_index("c")
        s = lax.axis_index("s")

        @pl.when(s == 0)
        def _():
            pltpu.sync_copy(zeros_hbm, acc_sh)

        plsc.subcore_barrier()
        base = s * _S_PER_S

        def _accum(val_hbm, i_hbm):
            @pl.loop(0, _S_PER_S // SUP)
            def _(i):
                b = base + i * SUP
                pltpu.sync_copy(
                    i_hbm.at[pl.ds(pl.multiple_of(b // 128, 8), SUPI), :],
                    idx_v)
                for half in range(SUP // SCH):
                    pltpu.sync_copy(
                        val_hbm.at[pl.ds(pl.multiple_of(b + half * SCH, 8), SCH)],
                        val_v)
                    for j in range(SCHI):
                        pltpu.sync_copy(val_v.at[pl.ds(j * 128, 128)],
                                        acc_sh.at[idx_v.at[half * SCHI + j]],
                                        add=True)

        @pl.when(c == 0)
        def _():
            _accum(sp_hbm, idxp_hbm)

        @pl.when(c == 1)
        def _():
            _accum(sc_hbm, idxc_hbm)

        plsc.subcore_barrier()

        @pl.when(s == 0)
        def _():
            pltpu.sync_copy(acc_sh, out_hbm.at[c])

    return _scatter_k(sp, sc, idxp2d, idxc2d, zeros)


def _sc_counts(idxp2d, idxc2d, ones, zeros):
    @functools.partial(
        pl.kernel, mesh=_mesh(),
        out_type=jax.ShapeDtypeStruct((NC, N_PAD, H2), F32),
        scratch_types=[pltpu.VMEM((SUPI, 128), jnp.int32),
                       pltpu.VMEM((128, H2), F32),
                       pltpu.VMEM_SHARED((N_PAD, H2), F32),
                       pltpu.SemaphoreType.DMA],
    )
    def _counts_k(idxp_hbm, idxc_hbm, ones_hbm, zeros_hbm, out_hbm,
                  idx_v, ones_v, acc_sh, sem):
        c = lax.axis_index("c")
        s = lax.axis_index("s")

        pltpu.sync_copy(ones_hbm, ones_v)

        @pl.when(s == 0)
        def _():
            pltpu.sync_copy(zeros_hbm, acc_sh)

        plsc.subcore_barrier()
        base = s * _S_PER_S

        def _accum(i_hbm):
            @pl.loop(0, _S_PER_S // SUP)
            def _(i):
                b = base + i * SUP
                pltpu.sync_copy(
                    i_hbm.at[pl.ds(pl.multiple_of(b // 128, 8), SUPI), :],
                    idx_v)
                for j in range(SUPI):
                    pltpu.sync_copy(ones_v, acc_sh.at[idx_v.at[j]], add=True)

        @pl.when(c == 0)
        def _():
            _accum(idxp_hbm)

        @pl.when(c == 1)
        def _():
            _accum(idxc_hbm)

        plsc.subcore_barrier()

        @pl.when(s == 0)
        def _():
            pltpu.sync_copy(acc_sh, out_hbm.at[c])

    return _counts_k(idxp2d, idxc2d, ones, zeros)


def _row(v):
    return v.reshape(1, -1).astype(F32)


def _prep2(p):
    return (p['W1'].T.astype(F32), _row(p['b1']), _row(p['g1']), _row(p['be1']),
            p['W2'].T.astype(F32), _row(p['b2']), _row(p['g2']), _row(p['be2']))


def _prep_e(p):
    return (p['W1'][:, 0].reshape(1, -1).astype(F32), _row(p['b1']),
            _row(p['g1']), _row(p['be1']),
            p['W2'].T.astype(F32), _row(p['b2']), _row(p['g2']), _row(p['be2']))


def _prep3(p, dt=F32):
    w1 = p['W1'].astype(dt)
    return (w1[:, :H2].T, w1[:, H2:2 * H2].T, w1[:, 2 * H2:].T,
            _row(p['b1']), _row(p['g1']), _row(p['be1']),
            p['W2'].T.astype(dt), _row(p['b2']), _row(p['g2']), _row(p['be2']))


def kernel(batch_token, edge_p_node, edge_c_node, edge_p_indicate,
           edge_c_indicate, p_mask, c_mask, start_token, end_token, params):
    bt = jnp.zeros((N_PAD, EMBED), F32).at[:N_NODES].set(batch_token)
    idx_p = jnp.full((E_PAD,), N_NODES, jnp.int32).at[:N_EDGES].set(
        edge_p_node.astype(jnp.int32))
    idx_c = jnp.full((E_PAD,), N_NODES, jnp.int32).at[:N_EDGES].set(
        edge_c_node.astype(jnp.int32))
    idx_all2d = jnp.concatenate([idx_p, idx_c]).reshape(E2 // 128, 128)
    idx_p2d = idx_p.reshape(E_PAD // 128, 128)
    idx_c2d = idx_c.reshape(E_PAD // 128, 128)

    ind = jnp.zeros((E2, 1), F32)
    ind = ind.at[:N_EDGES, 0].set(edge_p_indicate)
    ind = ind.at[E_PAD:E_PAD + N_EDGES, 0].set(edge_c_indicate)

    pm = jnp.zeros((N_PAD, 1), F32).at[:N_NODES, 0].set(p_mask)
    cm = jnp.zeros((N_PAD, 1), F32).at[:N_NODES, 0].set(c_mask)
    st = start_token.reshape(1, H2).astype(F32)
    en = end_token.reshape(1, H2).astype(F32)

    zN = jnp.zeros((N_PAD, H2), F32)
    ones128 = jnp.ones((128, H2), F32)

    wV = _prep2(params['V'])
    wE = _prep_e(params['E'])

    h = _node_mlp(bt, wV)
    ef = _edge_ind_mlp(ind, wE)
    ep, ec = ef[:E_PAD], ef[E_PAD:]

    counts = _sc_counts(idx_p2d, idx_c2d, ones128, zN)
    cp, cc = counts[0], counts[1]

    for hop in range(2):
        wp = _prep3(params['P'][hop], BF16)
        wc = _prep3(params['C'][hop], BF16)
        wa = _prep3(params['A'][hop])

        gath = _sc_gather(h, idx_all2d)
        pb, cb = gath[:E_PAD], gath[E_PAD:]
        sp, sc = _edge_mlp(pb, cb, ep, ec, wp, wc)
        sums = _sc_scatter(sp, sc, idx_p2d, idx_c2d, zN)
        h = _node_update(h, sums[0], sums[1], cp, cc, pm, cm, st, en, wa)

    return h[:N_NODES]


# speedup vs baseline: 2.3930x; 1.1716x over previous
import functools

import jax
import jax.numpy as jnp
from jax import lax
from jax.experimental import pallas as pl
from jax.experimental.pallas import tpu as pltpu
from jax.experimental.pallas import tpu_sc as plsc

F32 = jnp.float32
BF16 = jnp.bfloat16

N_NODES = 10000
N_EDGES = 320000
EMBED = 128
H1 = 256
H2 = 128

N_PAD = 10240
E_PAD = 327680
E2 = 2 * E_PAD

BN = 1024
BE = 2560

NC, NS = 2, 16
NW = NC * NS
CH = 512
CHI = CH // 128
SUP = 1024
SUPI = SUP // 128
SCH = 256
SCHI = SCH // 128


def _ln_relu(h, g, b):
    m = jnp.mean(h, axis=-1, keepdims=True)
    v = jnp.mean((h - m) ** 2, axis=-1, keepdims=True)
    return jnp.maximum((h - m) * lax.rsqrt(v + 1e-5) * g + b, 0.0)


def _dot(a, b):
    return lax.dot_general(a, b, (((1,), (0,)), ((), ())),
                           preferred_element_type=F32)


def _mlp2_body(x, w1t, b1, g1, be1, w2t, b2, g2, be2, o):
    h = _dot(x[...], w1t[...]) + b1[...]
    h = _ln_relu(h, g1[...], be1[...])
    h = _dot(h, w2t[...]) + b2[...]
    o[...] = _ln_relu(h, g2[...], be2[...])


def _emlp_body(x, w1r, b1, g1, be1, w2t, b2, g2, be2, o):
    h = x[...] * w1r[...] + b1[...]
    h = _ln_relu(h, g1[...], be1[...])
    h = _dot(h, w2t[...]) + b2[...]
    o[...] = _ln_relu(h, g2[...], be2[...]).astype(BF16)


def _edge_body(pb, cb, ep, ec,
               pat, pbt, pct, pb1, pg1, pbe1, pw2, pb2, pg2, pbe2,
               cat, cbt, cct, cb1, cg1, cbe1, cw2, cb2, cg2, cbe2,
               sp, sc):
    pbv, cbv = pb[...].astype(BF16), cb[...].astype(BF16)
    h = _dot(cbv, pat[...]) + _dot(pbv, pbt[...]) + _dot(ep[...], pct[...]) + pb1[...]
    h = _ln_relu(h, pg1[...], pbe1[...]).astype(BF16)
    h = _dot(h, pw2[...]) + pb2[...]
    sp[...] = _ln_relu(h, pg2[...], pbe2[...])

    h = _dot(pbv, cat[...]) + _dot(cbv, cbt[...]) + _dot(ec[...], cct[...]) + cb1[...]
    h = _ln_relu(h, cg1[...], cbe1[...]).astype(BF16)
    h = _dot(h, cw2[...]) + cb2[...]
    sc[...] = _ln_relu(h, cg2[...], cbe2[...])


def _update_body(h, sps, scs, cp, cc, pm, cm, st, en,
                 at_, bt_, ct_, b1, g1, be1, w2t, b2, g2, be2, o):
    hv = h[...]
    sp = sps[0] / jnp.maximum(cp[0][:, 0:1], 1.0) + pm[...] * st[...]
    sc = scs[0] / jnp.maximum(cc[0][:, 0:1], 1.0) + cm[...] * en[...]
    x = _dot(hv, at_[...]) + _dot(sp, bt_[...]) + _dot(sc, ct_[...]) + b1[...]
    x = _ln_relu(x, g1[...], be1[...])
    x = _dot(x, w2t[...]) + b2[...]
    x = _ln_relu(x, g2[...], be2[...])
    o[...] = jnp.maximum(hv + x, 0.0)


def _full(shape):
    return pl.BlockSpec(shape, lambda i: (0, 0))


_CPARAMS = pltpu.CompilerParams(dimension_semantics=("arbitrary",))


def _node_mlp(x, w):
    specs = [pl.BlockSpec((BN, EMBED), lambda i: (i, 0))]
    specs += [_full(t.shape) for t in w]
    return pl.pallas_call(
        _mlp2_body,
        grid=(N_PAD // BN,),
        in_specs=specs,
        out_specs=pl.BlockSpec((BN, H2), lambda i: (i, 0)),
        out_shape=jax.ShapeDtypeStruct((N_PAD, H2), F32),
        compiler_params=_CPARAMS,
    )(x, *w)


def _edge_ind_mlp(x, w):
    specs = [pl.BlockSpec((BE, 1), lambda i: (i, 0))]
    specs += [_full(t.shape) for t in w]
    return pl.pallas_call(
        _emlp_body,
        grid=(E2 // BE,),
        in_specs=specs,
        out_specs=pl.BlockSpec((BE, H2), lambda i: (i, 0)),
        out_shape=jax.ShapeDtypeStruct((E2, H2), BF16),
        compiler_params=_CPARAMS,
    )(x, *w)


def _edge_mlp(gath, ef, wp, wc):
    dspec = pl.BlockSpec((BE, H2), lambda i: (i, 0))
    hspec = pl.BlockSpec((BE, H2), lambda i: (i + E_PAD // BE, 0))
    specs = [dspec, hspec, dspec, hspec]
    specs += [_full(t.shape) for t in (*wp, *wc)]
    return pl.pallas_call(
        _edge_body,
        grid=(E_PAD // BE,),
        in_specs=specs,
        out_specs=[dspec, dspec],
        out_shape=[jax.ShapeDtypeStruct((E_PAD, H2), F32)] * 2,
        compiler_params=_CPARAMS,
    )(gath, gath, ef, ef, *wp, *wc)


def _node_update(h, sums, counts, pm, cm, st, en, wa):
    nspec = pl.BlockSpec((BN, H2), lambda i: (i, 0))
    cspec = pl.BlockSpec((BN, H2), lambda i: (i, 0))
    mspec = pl.BlockSpec((BN, 1), lambda i: (i, 0))
    tspec = _full((1, H2))
    s0spec = pl.BlockSpec((1, BN, H2), lambda i: (0, i, 0))
    s1spec = pl.BlockSpec((1, BN, H2), lambda i: (1, i, 0))
    c0spec = pl.BlockSpec((1, BN, H2), lambda i: (0, i, 0))
    c1spec = pl.BlockSpec((1, BN, H2), lambda i: (1, i, 0))
    specs = [nspec, s0spec, s1spec, c0spec, c1spec, mspec, mspec, tspec, tspec]
    specs += [_full(t.shape) for t in wa]
    return pl.pallas_call(
        _update_body,
        grid=(N_PAD // BN,),
        in_specs=specs,
        out_specs=nspec,
        out_shape=jax.ShapeDtypeStruct((N_PAD, H2), F32),
        compiler_params=_CPARAMS,
    )(h, sums, sums, counts, counts, pm, cm, st, en, *wa)


_G_PER_W = E2 // NW
_S_PER_S = E_PAD // NS


@functools.cache
def _mesh():
    return plsc.VectorSubcoreMesh(core_axis_name="c", subcore_axis_name="s")


def _sc_gather(h, idx2d):
    @functools.partial(
        pl.kernel, mesh=_mesh(),
        out_type=jax.ShapeDtypeStruct((E2, H2), F32),
        scratch_types=[pltpu.VMEM((SUPI, 128), jnp.int32),
                       pltpu.VMEM((CH, H2), F32),
                       pltpu.SemaphoreType.DMA],
    )
    def _gather_k(h_hbm, idx_hbm, out_hbm, idx_v, rows_v, sem):
        wid = lax.axis_index("s") * NC + lax.axis_index("c")
        base = wid * _G_PER_W

        @pl.loop(0, _G_PER_W // SUP)
        def _(i):
            b = base + i * SUP
            pltpu.sync_copy(
                idx_hbm.at[pl.ds(pl.multiple_of(b // 128, 8), SUPI), :], idx_v)
            for half in range(SUP // CH):
                cps = [pltpu.async_copy(h_hbm.at[idx_v.at[half * CHI + j]],
                                        rows_v.at[pl.ds(j * 128, 128)], sem)
                       for j in range(CHI)]
                for c in cps:
                    c.wait()
                pltpu.sync_copy(
                    rows_v,
                    out_hbm.at[pl.ds(pl.multiple_of(b + half * CH, 8), CH)])

    return _gather_k(h, idx2d)


def _sc_scatter(sp, sc, idxp2d, idxc2d, zeros):
    @functools.partial(
        pl.kernel, mesh=_mesh(),
        out_type=jax.ShapeDtypeStruct((NC, N_PAD, H2), F32),
        scratch_types=[pltpu.VMEM((SUPI, 128), jnp.int32),
                       pltpu.VMEM((SCH, H2), F32),
                       pltpu.VMEM_SHARED((N_PAD, H2), F32),
                       pltpu.SemaphoreType.DMA],
    )
    def _scatter_k(sp_hbm, sc_hbm, idxp_hbm, idxc_hbm, zeros_hbm, out_hbm,
                   idx_v, val_v, acc_sh, sem):
        c = lax.axis_index("c")
        s = lax.axis_index("s")

        @pl.when(s == 0)
        def _():
            pltpu.sync_copy(zeros_hbm, acc_sh)

        plsc.subcore_barrier()
        base = s * _S_PER_S

        def _accum(val_hbm, i_hbm):
            @pl.loop(0, _S_PER_S // SUP)
            def _(i):
                b = base + i * SUP
                pltpu.sync_copy(
                    i_hbm.at[pl.ds(pl.multiple_of(b // 128, 8), SUPI), :],
                    idx_v)
                for half in range(SUP // SCH):
                    pltpu.sync_copy(
                        val_hbm.at[pl.ds(pl.multiple_of(b + half * SCH, 8), SCH)],
                        val_v)
                    for j in range(SCHI):
                        pltpu.sync_copy(val_v.at[pl.ds(j * 128, 128)],
                                        acc_sh.at[idx_v.at[half * SCHI + j]],
                                        add=True)

        @pl.when(c == 0)
        def _():
            _accum(sp_hbm, idxp_hbm)

        @pl.when(c == 1)
        def _():
            _accum(sc_hbm, idxc_hbm)

        plsc.subcore_barrier()

        @pl.when(s == 0)
        def _():
            pltpu.sync_copy(acc_sh, out_hbm.at[c])

    return _scatter_k(sp, sc, idxp2d, idxc2d, zeros)


def _sc_counts(idxp2d, idxc2d, ones, zeros):
    @functools.partial(
        pl.kernel, mesh=_mesh(),
        out_type=jax.ShapeDtypeStruct((NC, N_PAD, H2), F32),
        scratch_types=[pltpu.VMEM((SUPI, 128), jnp.int32),
                       pltpu.VMEM((128, H2), F32),
                       pltpu.VMEM_SHARED((N_PAD, H2), F32),
                       pltpu.SemaphoreType.DMA],
    )
    def _counts_k(idxp_hbm, idxc_hbm, ones_hbm, zeros_hbm, out_hbm,
                  idx_v, ones_v, acc_sh, sem):
        c = lax.axis_index("c")
        s = lax.axis_index("s")

        pltpu.sync_copy(ones_hbm, ones_v)

        @pl.when(s == 0)
        def _():
            pltpu.sync_copy(zeros_hbm, acc_sh)

        plsc.subcore_barrier()
        base = s * _S_PER_S

        def _accum(i_hbm):
            @pl.loop(0, _S_PER_S // SUP)
            def _(i):
                b = base + i * SUP
                pltpu.sync_copy(
                    i_hbm.at[pl.ds(pl.multiple_of(b // 128, 8), SUPI), :],
                    idx_v)
                for j in range(SUPI):
                    pltpu.sync_copy(ones_v, acc_sh.at[idx_v.at[j]], add=True)

        @pl.when(c == 0)
        def _():
            _accum(idxp_hbm)

        @pl.when(c == 1)
        def _():
            _accum(idxc_hbm)

        plsc.subcore_barrier()

        @pl.when(s == 0)
        def _():
            pltpu.sync_copy(acc_sh, out_hbm.at[c])

    return _counts_k(idxp2d, idxc2d, ones, zeros)


def _row(v):
    return v.reshape(1, -1).astype(F32)


def _prep2(p):
    return (p['W1'].T.astype(F32), _row(p['b1']), _row(p['g1']), _row(p['be1']),
            p['W2'].T.astype(F32), _row(p['b2']), _row(p['g2']), _row(p['be2']))


def _prep_e(p):
    return (p['W1'][:, 0].reshape(1, -1).astype(F32), _row(p['b1']),
            _row(p['g1']), _row(p['be1']),
            p['W2'].T.astype(F32), _row(p['b2']), _row(p['g2']), _row(p['be2']))


def _prep3(p, dt=F32):
    w1 = p['W1'].astype(dt)
    return (w1[:, :H2].T, w1[:, H2:2 * H2].T, w1[:, 2 * H2:].T,
            _row(p['b1']), _row(p['g1']), _row(p['be1']),
            p['W2'].T.astype(dt), _row(p['b2']), _row(p['g2']), _row(p['be2']))


def kernel(batch_token, edge_p_node, edge_c_node, edge_p_indicate,
           edge_c_indicate, p_mask, c_mask, start_token, end_token, params):
    bt = jnp.concatenate(
        [batch_token, jnp.zeros((N_PAD - N_NODES, EMBED), F32)])
    ipad = jnp.full((E_PAD - N_EDGES,), N_NODES, jnp.int32)
    idx_p = jnp.concatenate([edge_p_node.astype(jnp.int32), ipad])
    idx_c = jnp.concatenate([edge_c_node.astype(jnp.int32), ipad])
    idx_all2d = jnp.concatenate([idx_p, idx_c]).reshape(E2 // 128, 128)
    idx_p2d = idx_p.reshape(E_PAD // 128, 128)
    idx_c2d = idx_c.reshape(E_PAD // 128, 128)

    fpad = jnp.zeros((E_PAD - N_EDGES,), F32)
    ind = jnp.concatenate(
        [edge_p_indicate, fpad, edge_c_indicate, fpad]).reshape(E2, 1)

    pm = jnp.concatenate([p_mask, jnp.zeros((N_PAD - N_NODES,), F32)]
                         ).reshape(N_PAD, 1)
    cm = jnp.concatenate([c_mask, jnp.zeros((N_PAD - N_NODES,), F32)]
                         ).reshape(N_PAD, 1)
    st = start_token.reshape(1, H2).astype(F32)
    en = end_token.reshape(1, H2).astype(F32)

    zN = jnp.zeros((N_PAD, H2), F32)
    ones128 = jnp.ones((128, H2), F32)

    wV = _prep2(params['V'])
    wE = _prep_e(params['E'])

    h = _node_mlp(bt, wV)
    ef = _edge_ind_mlp(ind, wE)

    counts = _sc_counts(idx_p2d, idx_c2d, ones128, zN)

    for hop in range(2):
        wp = _prep3(params['P'][hop], BF16)
        wc = _prep3(params['C'][hop], BF16)
        wa = _prep3(params['A'][hop])

        gath = _sc_gather(h, idx_all2d)
        sp, sc = _edge_mlp(gath, ef, wp, wc)
        sums = _sc_scatter(sp, sc, idx_p2d, idx_c2d, zN)
        h = _node_update(h, sums, counts, pm, cm, st, en, wa)

    return h[:N_NODES]
